# baseline (speedup 1.0000x reference)
import functools

import jax
import jax.numpy as jnp
from jax import lax
from jax.experimental import pallas as pl
from jax.experimental.pallas import tpu as pltpu
from jax.experimental.pallas import tpu_sc as plsc

N_NODES = 10000
N_EDGES = 320000
D_FEAT = 128

NC = 2
NS = 16
NW = NC * NS
EDGES_PER_TILE = N_EDGES // NW
CHUNK = 80
NCHUNKS = EDGES_PER_TILE // CHUNK
ROWS_PER_TILE = 640
N_PAD = NS * ROWS_PER_TILE


def _sc_segment_sum(x, src, dst, zeros):
    mesh = plsc.VectorSubcoreMesh(core_axis_name="c", subcore_axis_name="s")

    @functools.partial(
        pl.kernel,
        mesh=mesh,
        out_type=jax.ShapeDtypeStruct((NC, N_PAD, D_FEAT), jnp.float32),
        scratch_types=[
            pltpu.VMEM_SHARED((N_PAD, D_FEAT), jnp.float32),
            pltpu.VMEM((EDGES_PER_TILE,), jnp.int32),
            pltpu.VMEM((CHUNK,), jnp.int32),
            pltpu.VMEM((CHUNK,), jnp.int32),
            pltpu.VMEM((CHUNK, D_FEAT), jnp.float32),
            pltpu.VMEM((CHUNK, D_FEAT), jnp.float32),
            pltpu.SemaphoreType.DMA,
            pltpu.SemaphoreType.DMA,
            pltpu.SemaphoreType.DMA,
            pltpu.SemaphoreType.DMA,
            pltpu.SemaphoreType.DMA,
            pltpu.SemaphoreType.DMA,
        ],
    )
    def k(x_hbm, src_hbm, dst_hbm, zeros_hbm, out_hbm, acc_sh, sidx, didx0,
          didx1, rows0, rows1, sem0, sem1, semd0, semd1, sems0, sems1):
        cid = lax.axis_index("c")
        sid = lax.axis_index("s")
        wid = cid * NS + sid
        slab = pl.ds(sid * ROWS_PER_TILE, ROWS_PER_TILE)
        ebase = wid * EDGES_PER_TILE
        pltpu.sync_copy(src_hbm.at[pl.ds(ebase, EDGES_PER_TILE)], sidx)
        pltpu.sync_copy(zeros_hbm.at[slab], acc_sh.at[slab])
        plsc.subcore_barrier()

        def sidx_c(i):
            return sidx.at[pl.ds(i * CHUNK, CHUNK)]

        def didx_c(i):
            return dst_hbm.at[pl.ds(ebase + i * CHUNK, CHUNK)]

        def g_wait(i, rows, sem):
            pltpu.make_async_copy(x_hbm.at[sidx_c(i)], rows, sem).wait()

        def d_wait(i, didx, semd):
            pltpu.make_async_copy(didx_c(i), didx, semd).wait()

        def s_wait(rows, didx, sems):
            pltpu.make_async_copy(rows, acc_sh.at[didx], sems).wait()

        pltpu.async_copy(didx_c(0), didx0, semd0)
        pltpu.async_copy(x_hbm.at[sidx_c(0)], rows0, sem0)
        pltpu.async_copy(didx_c(1), didx1, semd1)
        pltpu.async_copy(x_hbm.at[sidx_c(1)], rows1, sem1)

        def body(j, carry):
            i = 2 * j
            g_wait(i, rows0, sem0)
            d_wait(i, didx0, semd0)
            pltpu.async_copy(rows0, acc_sh.at[didx0], sems0, add=True)
            g_wait(i + 1, rows1, sem1)
            d_wait(i + 1, didx1, semd1)
            pltpu.async_copy(rows1, acc_sh.at[didx1], sems1, add=True)
            s_wait(rows0, didx0, sems0)
            pltpu.async_copy(didx_c(i + 2), didx0, semd0)
            pltpu.async_copy(x_hbm.at[sidx_c(i + 2)], rows0, sem0)
            s_wait(rows1, didx1, sems1)
            pltpu.async_copy(didx_c(i + 3), didx1, semd1)
            pltpu.async_copy(x_hbm.at[sidx_c(i + 3)], rows1, sem1)
            return carry

        lax.fori_loop(0, (NCHUNKS - 3) // 2, body, 0)
        g_wait(NCHUNKS - 3, rows0, sem0)
        d_wait(NCHUNKS - 3, didx0, semd0)
        pltpu.async_copy(rows0, acc_sh.at[didx0], sems0, add=True)
        g_wait(NCHUNKS - 2, rows1, sem1)
        d_wait(NCHUNKS - 2, didx1, semd1)
        pltpu.async_copy(rows1, acc_sh.at[didx1], sems1, add=True)
        s_wait(rows0, didx0, sems0)
        pltpu.async_copy(didx_c(NCHUNKS - 1), didx0, semd0)
        pltpu.async_copy(x_hbm.at[sidx_c(NCHUNKS - 1)], rows0, sem0)
        g_wait(NCHUNKS - 1, rows0, sem0)
        d_wait(NCHUNKS - 1, didx0, semd0)
        pltpu.async_copy(rows0, acc_sh.at[didx0], sems0, add=True)
        s_wait(rows0, didx0, sems0)
        s_wait(rows1, didx1, sems1)
        plsc.subcore_barrier()
        pltpu.sync_copy(acc_sh.at[slab], out_hbm.at[cid, slab])

    return k(x, src, dst, zeros)


BLK_ROWS = 1000


def _tc_xs(x, Wst):
    def body(x_ref, wst_ref, o_ref):
        o_ref[...] = jnp.dot(x_ref[...], wst_ref[...],
                             preferred_element_type=jnp.float32)

    return pl.pallas_call(
        body,
        grid=(N_NODES // BLK_ROWS,),
        in_specs=[
            pl.BlockSpec((BLK_ROWS, D_FEAT), lambda i: (i, 0)),
            pl.BlockSpec((D_FEAT, D_FEAT), lambda i: (0, 0)),
        ],
        out_specs=pl.BlockSpec((BLK_ROWS, D_FEAT), lambda i: (i, 0)),
        out_shape=jax.ShapeDtypeStruct((N_NODES, D_FEAT), jnp.float32),
    )(x, Wst)


def _tc_final(xs, partials, Wnt):
    def body(xs_ref, p_ref, wnt_ref, o_ref):
        agg = p_ref[0] + p_ref[1]
        z = xs_ref[...] + jnp.dot(agg, wnt_ref[...],
                                  preferred_element_type=jnp.float32)
        o_ref[...] = 0.5 * z * (1.0 + lax.erf(z * 0.7071067811865476))

    return pl.pallas_call(
        body,
        grid=(N_NODES // BLK_ROWS,),
        in_specs=[
            pl.BlockSpec((BLK_ROWS, D_FEAT), lambda i: (i, 0)),
            pl.BlockSpec((NC, BLK_ROWS, D_FEAT), lambda i: (0, i, 0)),
            pl.BlockSpec((D_FEAT, D_FEAT), lambda i: (0, 0)),
        ],
        out_specs=pl.BlockSpec((BLK_ROWS, D_FEAT), lambda i: (i, 0)),
        out_shape=jax.ShapeDtypeStruct((N_NODES, D_FEAT), jnp.float32),
    )(xs, partials, Wnt)


def kernel(x, edge_index, W_s, W_n):
    src = edge_index[0].astype(jnp.int32)
    dst = edge_index[1].astype(jnp.int32)
    zeros = jnp.zeros((N_PAD, D_FEAT), jnp.float32)
    xs = _tc_xs(x, W_s.T)
    partials = _sc_segment_sum(x, src, dst, zeros)
    return _tc_final(xs, partials, W_n.T)


# speedup vs baseline: 1.2182x; 1.2182x over previous
import functools

import jax
import jax.numpy as jnp
from jax import lax
from jax.experimental import pallas as pl
from jax.experimental.pallas import tpu as pltpu
from jax.experimental.pallas import tpu_sc as plsc

N_NODES = 10000
N_EDGES = 320000
D_FEAT = 128

NC = 2
NS = 16
NW = NC * NS
EDGES_PER_TILE = N_EDGES // NW
CHUNK = 80
NCHUNKS = EDGES_PER_TILE // CHUNK
ROWS_PER_TILE = 640
N_PAD = NS * ROWS_PER_TILE


def _sc_segment_sum(x, src, dst, zeros):
    mesh = plsc.VectorSubcoreMesh(core_axis_name="c", subcore_axis_name="s")

    @functools.partial(
        pl.kernel,
        mesh=mesh,
        out_type=jax.ShapeDtypeStruct((NC, N_PAD, D_FEAT), jnp.float32),
        scratch_types=[
            pltpu.VMEM_SHARED((N_PAD, D_FEAT), jnp.float32),
            pltpu.VMEM((EDGES_PER_TILE,), jnp.int32),
            pltpu.VMEM((CHUNK,), jnp.int32),
            pltpu.VMEM((CHUNK,), jnp.int32),
            pltpu.VMEM((CHUNK, D_FEAT), jnp.float32),
            pltpu.VMEM((CHUNK, D_FEAT), jnp.float32),
            pltpu.SemaphoreType.DMA,
            pltpu.SemaphoreType.DMA,
            pltpu.SemaphoreType.DMA,
            pltpu.SemaphoreType.DMA,
            pltpu.SemaphoreType.DMA,
            pltpu.SemaphoreType.DMA,
        ],
    )
    def k(x_hbm, src_hbm, dst_hbm, zeros_hbm, out_hbm, acc_sh, sidx, didx0,
          didx1, rows0, rows1, sem0, sem1, semd0, semd1, sems0, sems1):
        cid = lax.axis_index("c")
        sid = lax.axis_index("s")
        wid = cid * NS + sid
        slab = pl.ds(sid * ROWS_PER_TILE, ROWS_PER_TILE)
        ebase = wid * EDGES_PER_TILE
        pltpu.sync_copy(src_hbm.at[pl.ds(ebase, EDGES_PER_TILE)], sidx)
        pltpu.sync_copy(zeros_hbm.at[slab], acc_sh.at[slab])
        plsc.subcore_barrier()

        def sidx_c(i):
            return sidx.at[pl.ds(i * CHUNK, CHUNK)]

        def didx_c(i):
            return dst_hbm.at[pl.ds(ebase + i * CHUNK, CHUNK)]

        pltpu.async_copy(didx_c(0), didx0, semd0)
        pltpu.async_copy(x_hbm.at[sidx_c(0)], rows0, sem0)

        def body(j, carry):
            i = 2 * j
            pltpu.async_copy(didx_c(i + 1), didx1, semd1)
            pltpu.async_copy(x_hbm.at[sidx_c(i + 1)], rows1, sem1)
            pltpu.make_async_copy(x_hbm.at[sidx_c(i)], rows0, sem0).wait()
            pltpu.make_async_copy(didx_c(i), didx0, semd0).wait()
            pltpu.sync_copy(rows0, acc_sh.at[didx0], add=True)
            pltpu.async_copy(didx_c(i + 2), didx0, semd0)
            pltpu.async_copy(x_hbm.at[sidx_c(i + 2)], rows0, sem0)
            pltpu.make_async_copy(x_hbm.at[sidx_c(i + 1)], rows1, sem1).wait()
            pltpu.make_async_copy(didx_c(i + 1), didx1, semd1).wait()
            pltpu.sync_copy(rows1, acc_sh.at[didx1], add=True)
            return carry

        lax.fori_loop(0, (NCHUNKS - 1) // 2, body, 0)
        pltpu.make_async_copy(x_hbm.at[sidx_c(NCHUNKS - 1)], rows0,
                              sem0).wait()
        pltpu.make_async_copy(didx_c(NCHUNKS - 1), didx0, semd0).wait()
        pltpu.sync_copy(rows0, acc_sh.at[didx0], add=True)
        plsc.subcore_barrier()
        pltpu.sync_copy(acc_sh.at[slab], out_hbm.at[cid, slab])

    return k(x, src, dst, zeros)


BLK_ROWS = 1000


def _tc_xs(x, Wst):
    def body(x_ref, wst_ref, o_ref):
        o_ref[...] = jnp.dot(x_ref[...], wst_ref[...],
                             preferred_element_type=jnp.float32)

    return pl.pallas_call(
        body,
        grid=(N_NODES // BLK_ROWS,),
        in_specs=[
            pl.BlockSpec((BLK_ROWS, D_FEAT), lambda i: (i, 0)),
            pl.BlockSpec((D_FEAT, D_FEAT), lambda i: (0, 0)),
        ],
        out_specs=pl.BlockSpec((BLK_ROWS, D_FEAT), lambda i: (i, 0)),
        out_shape=jax.ShapeDtypeStruct((N_NODES, D_FEAT), jnp.float32),
    )(x, Wst)


def _tc_final(xs, partials, Wnt):
    def body(xs_ref, p_ref, wnt_ref, o_ref):
        agg = p_ref[0] + p_ref[1]
        z = xs_ref[...] + jnp.dot(agg, wnt_ref[...],
                                  preferred_element_type=jnp.float32)
        o_ref[...] = 0.5 * z * (1.0 + lax.erf(z * 0.7071067811865476))

    return pl.pallas_call(
        body,
        grid=(N_NODES // BLK_ROWS,),
        in_specs=[
            pl.BlockSpec((BLK_ROWS, D_FEAT), lambda i: (i, 0)),
            pl.BlockSpec((NC, BLK_ROWS, D_FEAT), lambda i: (0, i, 0)),
            pl.BlockSpec((D_FEAT, D_FEAT), lambda i: (0, 0)),
        ],
        out_specs=pl.BlockSpec((BLK_ROWS, D_FEAT), lambda i: (i, 0)),
        out_shape=jax.ShapeDtypeStruct((N_NODES, D_FEAT), jnp.float32),
    )(xs, partials, Wnt)


def kernel(x, edge_index, W_s, W_n):
    src = edge_index[0].astype(jnp.int32)
    dst = edge_index[1].astype(jnp.int32)
    zeros = jnp.zeros((N_PAD, D_FEAT), jnp.float32)
    xs = _tc_xs(x, W_s.T)
    partials = _sc_segment_sum(x, src, dst, zeros)
    return _tc_final(xs, partials, W_n.T)


# speedup vs baseline: 1.3029x; 1.0696x over previous
import functools

import jax
import jax.numpy as jnp
from jax import lax
from jax.experimental import pallas as pl
from jax.experimental.pallas import tpu as pltpu
from jax.experimental.pallas import tpu_sc as plsc

N_NODES = 10000
N_EDGES = 320000
D_FEAT = 128

NC = 2
NS = 16
NW = NC * NS
EDGES_PER_TILE = N_EDGES // NW
CHUNK = 80
NCHUNKS = EDGES_PER_TILE // CHUNK
ROWS_PER_TILE = 640
N_PAD = NS * ROWS_PER_TILE


def _sc_segment_sum(x, edges, zeros):
    mesh = plsc.VectorSubcoreMesh(core_axis_name="c", subcore_axis_name="s")

    @functools.partial(
        pl.kernel,
        mesh=mesh,
        out_type=jax.ShapeDtypeStruct((NC, N_PAD, D_FEAT), jnp.float32),
        scratch_types=[
            pltpu.VMEM_SHARED((N_PAD, D_FEAT), jnp.float32),
            pltpu.VMEM((EDGES_PER_TILE,), jnp.int32),
            pltpu.VMEM((CHUNK,), jnp.int32),
            pltpu.VMEM((CHUNK,), jnp.int32),
            pltpu.VMEM((CHUNK, D_FEAT), jnp.float32),
            pltpu.VMEM((CHUNK, D_FEAT), jnp.float32),
            pltpu.SemaphoreType.DMA,
            pltpu.SemaphoreType.DMA,
            pltpu.SemaphoreType.DMA,
            pltpu.SemaphoreType.DMA,
            pltpu.SemaphoreType.DMA,
        ],
    )
    def k(x_hbm, edges_hbm, zeros_hbm, out_hbm, acc_sh, sidx, didx0,
          didx1, rows0, rows1, semi, sem0, sem1, semd0, semd1):
        cid = lax.axis_index("c")
        sid = lax.axis_index("s")
        wid = cid * NS + sid
        slab = pl.ds(sid * ROWS_PER_TILE, ROWS_PER_TILE)
        ebase = wid * EDGES_PER_TILE

        def didx_c(i):
            return edges_hbm.at[pl.ds(N_EDGES + ebase + i * CHUNK, CHUNK)]

        pltpu.async_copy(didx_c(0), didx0, semd0)
        pltpu.async_copy(edges_hbm.at[pl.ds(ebase, EDGES_PER_TILE)], sidx,
                         semi)
        pltpu.sync_copy(zeros_hbm, acc_sh.at[slab])
        pltpu.make_async_copy(edges_hbm.at[pl.ds(ebase, EDGES_PER_TILE)],
                              sidx, semi).wait()

        def sidx_c(i):
            return sidx.at[pl.ds(i * CHUNK, CHUNK)]

        pltpu.async_copy(x_hbm.at[sidx_c(0)], rows0, sem0)
        plsc.subcore_barrier()

        def body(j, carry):
            i = 2 * j
            pltpu.async_copy(didx_c(i + 1), didx1, semd1)
            pltpu.async_copy(x_hbm.at[sidx_c(i + 1)], rows1, sem1)
            pltpu.make_async_copy(x_hbm.at[sidx_c(i)], rows0, sem0).wait()
            pltpu.make_async_copy(didx_c(i), didx0, semd0).wait()
            pltpu.sync_copy(rows0, acc_sh.at[didx0], add=True)
            pltpu.async_copy(didx_c(i + 2), didx0, semd0)
            pltpu.async_copy(x_hbm.at[sidx_c(i + 2)], rows0, sem0)
            pltpu.make_async_copy(x_hbm.at[sidx_c(i + 1)], rows1, sem1).wait()
            pltpu.make_async_copy(didx_c(i + 1), didx1, semd1).wait()
            pltpu.sync_copy(rows1, acc_sh.at[didx1], add=True)
            return carry

        lax.fori_loop(0, (NCHUNKS - 1) // 2, body, 0)
        pltpu.make_async_copy(x_hbm.at[sidx_c(NCHUNKS - 1)], rows0,
                              sem0).wait()
        pltpu.make_async_copy(didx_c(NCHUNKS - 1), didx0, semd0).wait()
        pltpu.sync_copy(rows0, acc_sh.at[didx0], add=True)
        plsc.subcore_barrier()
        pltpu.sync_copy(acc_sh.at[slab], out_hbm.at[cid, slab])

    return k(x, edges, zeros)


BLK_ROWS = 1000


def _tc_xs(x, Wst):
    def body(x_ref, wst_ref, o_ref):
        o_ref[...] = jnp.dot(x_ref[...], wst_ref[...],
                             preferred_element_type=jnp.float32)

    return pl.pallas_call(
        body,
        grid=(N_NODES // BLK_ROWS,),
        in_specs=[
            pl.BlockSpec((BLK_ROWS, D_FEAT), lambda i: (i, 0)),
            pl.BlockSpec((D_FEAT, D_FEAT), lambda i: (0, 0)),
        ],
        out_specs=pl.BlockSpec((BLK_ROWS, D_FEAT), lambda i: (i, 0)),
        out_shape=jax.ShapeDtypeStruct((N_NODES, D_FEAT), jnp.float32),
    )(x, Wst)


def _tc_final(xs, partials, Wnt):
    def body(xs_ref, p_ref, wnt_ref, o_ref):
        agg = p_ref[0] + p_ref[1]
        z = xs_ref[...] + jnp.dot(agg, wnt_ref[...],
                                  preferred_element_type=jnp.float32)
        o_ref[...] = 0.5 * z * (1.0 + lax.erf(z * 0.7071067811865476))

    return pl.pallas_call(
        body,
        grid=(N_NODES // BLK_ROWS,),
        in_specs=[
            pl.BlockSpec((BLK_ROWS, D_FEAT), lambda i: (i, 0)),
            pl.BlockSpec((NC, BLK_ROWS, D_FEAT), lambda i: (0, i, 0)),
            pl.BlockSpec((D_FEAT, D_FEAT), lambda i: (0, 0)),
        ],
        out_specs=pl.BlockSpec((BLK_ROWS, D_FEAT), lambda i: (i, 0)),
        out_shape=jax.ShapeDtypeStruct((N_NODES, D_FEAT), jnp.float32),
    )(xs, partials, Wnt)


def kernel(x, edge_index, W_s, W_n):
    edges = edge_index.astype(jnp.int32).reshape(2 * N_EDGES)
    zeros = jnp.zeros((ROWS_PER_TILE, D_FEAT), jnp.float32)
    xs = _tc_xs(x, W_s.T)
    partials = _sc_segment_sum(x, edges, zeros)
    return _tc_final(xs, partials, W_n.T)


# speedup vs baseline: 1.4772x; 1.1338x over previous
import functools

import jax
import jax.numpy as jnp
from jax import lax
from jax.experimental import pallas as pl
from jax.experimental.pallas import tpu as pltpu
from jax.experimental.pallas import tpu_sc as plsc

N_NODES = 10000
N_EDGES = 320000
D_FEAT = 128

NC = 2
NS = 16
NW = NC * NS
EDGES_PER_TILE = N_EDGES // NW
CHUNK = 64
NFULL = EDGES_PER_TILE // CHUNK
TAIL = EDGES_PER_TILE - NFULL * CHUNK
ROWS_PER_TILE = 640
N_PAD = NS * ROWS_PER_TILE


def _sc_segment_sum(x, edges, zeros):
    mesh = plsc.VectorSubcoreMesh(core_axis_name="c", subcore_axis_name="s")

    @functools.partial(
        pl.kernel,
        mesh=mesh,
        out_type=jax.ShapeDtypeStruct((NC, N_PAD, D_FEAT), jnp.float32),
        scratch_types=[
            pltpu.VMEM_SHARED((N_PAD, D_FEAT), jnp.float32),
            pltpu.VMEM((EDGES_PER_TILE,), jnp.int32),
            pltpu.VMEM((CHUNK,), jnp.int32),
            pltpu.VMEM((CHUNK,), jnp.int32),
            pltpu.VMEM((CHUNK,), jnp.int32),
            pltpu.VMEM((TAIL,), jnp.int32),
            pltpu.VMEM((CHUNK, D_FEAT), jnp.float32),
            pltpu.VMEM((CHUNK, D_FEAT), jnp.float32),
            pltpu.VMEM((CHUNK, D_FEAT), jnp.float32),
            pltpu.SemaphoreType.DMA,
            pltpu.SemaphoreType.DMA,
            pltpu.SemaphoreType.DMA,
            pltpu.SemaphoreType.DMA,
            pltpu.SemaphoreType.DMA,
            pltpu.SemaphoreType.DMA,
            pltpu.SemaphoreType.DMA,
        ],
    )
    def k(x_hbm, edges_hbm, zeros_hbm, out_hbm, acc_sh, sidx, dd0, dd1, dd2,
          ddt, r0, r1, r2, semi, g0, g1, g2, d0, d1, d2):
        cid = lax.axis_index("c")
        sid = lax.axis_index("s")
        wid = cid * NS + sid
        slab = pl.ds(sid * ROWS_PER_TILE, ROWS_PER_TILE)
        ebase = wid * EDGES_PER_TILE

        def didx_c(i):
            return edges_hbm.at[pl.ds(N_EDGES + ebase + i * CHUNK, CHUNK)]

        def sidx_c(i):
            return sidx.at[pl.ds(i * CHUNK, CHUNK)]

        pltpu.async_copy(didx_c(0), dd0, d0)
        pltpu.async_copy(didx_c(1), dd1, d1)
        pltpu.async_copy(didx_c(2), dd2, d2)
        pltpu.async_copy(edges_hbm.at[pl.ds(ebase, EDGES_PER_TILE)], sidx,
                         semi)
        pltpu.sync_copy(zeros_hbm, acc_sh.at[slab])
        pltpu.make_async_copy(edges_hbm.at[pl.ds(ebase, EDGES_PER_TILE)],
                              sidx, semi).wait()

        pltpu.async_copy(x_hbm.at[sidx_c(0)], r0, g0)
        pltpu.async_copy(x_hbm.at[sidx_c(1)], r1, g1)
        pltpu.async_copy(x_hbm.at[sidx_c(2)], r2, g2)
        plsc.subcore_barrier()

        bufs = ((r0, dd0, g0, d0), (r1, dd1, g1, d1), (r2, dd2, g2, d2))

        def stage(c, rb, ddb, gb, db, launch):
            pltpu.make_async_copy(x_hbm.at[sidx_c(c)], rb, gb).wait()
            pltpu.make_async_copy(didx_c(c), ddb, db).wait()
            pltpu.sync_copy(rb, acc_sh.at[ddb], add=True)
            if launch:
                pltpu.async_copy(didx_c(c + 3), ddb, db)
                pltpu.async_copy(x_hbm.at[sidx_c(c + 3)], rb, gb)

        def body(t, carry):
            i = 3 * t
            for b, (rb, ddb, gb, db) in enumerate(bufs):
                stage(i + b, rb, ddb, gb, db, True)
            return carry

        lax.fori_loop(0, NFULL // 3 - 1, body, 0)
        stage(NFULL - 3, r0, dd0, g0, d0, False)
        tail_off = ebase + NFULL * CHUNK
        pltpu.async_copy(edges_hbm.at[pl.ds(N_EDGES + tail_off, TAIL)],
                         ddt, d0)
        pltpu.async_copy(x_hbm.at[sidx.at[pl.ds(NFULL * CHUNK, TAIL)]],
                         r0.at[pl.ds(0, TAIL)], g0)
        stage(NFULL - 2, r1, dd1, g1, d1, False)
        stage(NFULL - 1, r2, dd2, g2, d2, False)
        pltpu.make_async_copy(x_hbm.at[sidx.at[pl.ds(NFULL * CHUNK, TAIL)]],
                              r0.at[pl.ds(0, TAIL)], g0).wait()
        pltpu.make_async_copy(edges_hbm.at[pl.ds(N_EDGES + tail_off, TAIL)],
                              ddt, d0).wait()
        pltpu.sync_copy(r0.at[pl.ds(0, TAIL)], acc_sh.at[ddt], add=True)
        plsc.subcore_barrier()
        pltpu.sync_copy(acc_sh.at[slab], out_hbm.at[cid, slab])

    return k(x, edges, zeros)


BLK_ROWS = 1000


def _tc_xs(x, Wst):
    def body(x_ref, wst_ref, o_ref):
        o_ref[...] = jnp.dot(x_ref[...], wst_ref[...],
                             preferred_element_type=jnp.float32)

    return pl.pallas_call(
        body,
        grid=(N_NODES // BLK_ROWS,),
        in_specs=[
            pl.BlockSpec((BLK_ROWS, D_FEAT), lambda i: (i, 0)),
            pl.BlockSpec((D_FEAT, D_FEAT), lambda i: (0, 0)),
        ],
        out_specs=pl.BlockSpec((BLK_ROWS, D_FEAT), lambda i: (i, 0)),
        out_shape=jax.ShapeDtypeStruct((N_NODES, D_FEAT), jnp.float32),
    )(x, Wst)


def _tc_final(xs, partials, Wnt):
    def body(xs_ref, p_ref, wnt_ref, o_ref):
        agg = p_ref[0] + p_ref[1]
        z = xs_ref[...] + jnp.dot(agg, wnt_ref[...],
                                  preferred_element_type=jnp.float32)
        o_ref[...] = 0.5 * z * (1.0 + lax.erf(z * 0.7071067811865476))

    return pl.pallas_call(
        body,
        grid=(N_NODES // BLK_ROWS,),
        in_specs=[
            pl.BlockSpec((BLK_ROWS, D_FEAT), lambda i: (i, 0)),
            pl.BlockSpec((NC, BLK_ROWS, D_FEAT), lambda i: (0, i, 0)),
            pl.BlockSpec((D_FEAT, D_FEAT), lambda i: (0, 0)),
        ],
        out_specs=pl.BlockSpec((BLK_ROWS, D_FEAT), lambda i: (i, 0)),
        out_shape=jax.ShapeDtypeStruct((N_NODES, D_FEAT), jnp.float32),
    )(xs, partials, Wnt)


def kernel(x, edge_index, W_s, W_n):
    edges = edge_index.astype(jnp.int32).reshape(2 * N_EDGES)
    zeros = jnp.zeros((ROWS_PER_TILE, D_FEAT), jnp.float32)
    xs = _tc_xs(x, W_s.T)
    partials = _sc_segment_sum(x, edges, zeros)
    return _tc_final(xs, partials, W_n.T)
